# baseline (speedup 1.0000x reference)
import functools

import jax
import jax.numpy as jnp
import numpy as np
from jax import lax
from jax.experimental import pallas as pl
from jax.experimental.pallas import tpu as pltpu
from jax.experimental.pallas import tpu_sc as plsc

SEQLEN = 2048
D_MODEL = 768
BATCH = 4
SCALE = float(np.sqrt(float(D_MODEL)))

NC, NS, L = 2, 16, 16
NW = NC * NS
TOTAL = BATCH * SEQLEN
ROWS_PER_W = TOTAL // NW
CHUNK = 64
NCHUNK = ROWS_PER_W // CHUNK
SEQ_PER_W = SEQLEN // (NW // BATCH)


def _position_encoding(seqlen, d_model, times=10000):
    pos = np.arange(seqlen)[:, np.newaxis].astype(np.float64)
    depths = np.arange(d_model)[np.newaxis, :].astype(np.float64)
    depths = 2 * (depths // 2) / d_model
    angle_rates = 1.0 / times ** depths
    angle_rads = pos * angle_rates
    pe = np.zeros((seqlen, d_model), dtype=np.float64)
    pe[:, 0::2] = np.sin(angle_rads)[:, 0::2]
    pe[:, 1::2] = np.cos(angle_rads)[:, 1::2]
    return pe.astype(np.float32)


_PE = _position_encoding(SEQLEN, D_MODEL)

_mesh = plsc.VectorSubcoreMesh(core_axis_name="c", subcore_axis_name="s")


@functools.partial(
    pl.kernel,
    mesh=_mesh,
    out_type=jax.ShapeDtypeStruct((TOTAL, D_MODEL), jnp.float32),
    scratch_types=[
        pltpu.VMEM((NCHUNK, CHUNK), jnp.int32),
        pltpu.VMEM((CHUNK, D_MODEL), jnp.float32),
        pltpu.VMEM((CHUNK, D_MODEL), jnp.float32),
        pltpu.SemaphoreType.DMA,
    ],
)
def _emb(x_hbm, pe_hbm, table_hbm, out_hbm, idx_v, rows_v, pe_v, sem):
    wid = lax.axis_index("s") * NC + lax.axis_index("c")
    base = wid * ROWS_PER_W
    s_base = lax.rem(base, SEQLEN)
    pltpu.sync_copy(x_hbm.at[wid], idx_v)
    for c in range(NCHUNK):
        pltpu.async_copy(table_hbm.at[idx_v.at[c]], rows_v, sem).wait()
        pltpu.sync_copy(pe_hbm.at[pl.ds(s_base + c * CHUNK, CHUNK)], pe_v)

        def row_body(i, _):
            for j in range(D_MODEL // L):
                sl = pl.ds(j * L, L)
                rows_v[i, sl] = rows_v[i, sl] * SCALE + pe_v[i, sl]
            return _

        lax.fori_loop(0, CHUNK, row_body, None)
        pltpu.sync_copy(rows_v, out_hbm.at[pl.ds(base + c * CHUNK, CHUNK)])


def kernel(x, table):
    idx = x.astype(jnp.int32).reshape(NW, NCHUNK, CHUNK)
    out = _emb(idx, _PE, table)
    return out.reshape(BATCH, SEQLEN, D_MODEL)
